# Initial kernel scaffold; baseline (speedup 1.0000x reference)
import jax
import jax.numpy as jnp
from jax.experimental import pallas as pl


def kernel(x, Wg1, bg1, Wg2, bg2, We1, be1, We2, be2, Wc1, bc1, Wc2, bc2):
    raise NotImplementedError("write your pallas kernel here")


# speedup vs baseline: 3.2146x; 3.2146x over previous
import functools

import jax
import jax.numpy as jnp
from jax.experimental import pallas as pl
from jax.experimental.pallas import tpu as pltpu

B = 1024
D = 1024
DFF = 1024
E = 8
K = 2
GH = 512
CH = 512
OUT = 3


def _gating_body(x_ref, Wg1_ref, bg1_ref, Wg2_ref, bg2_ref, probs_ref, w_ref):
    x = x_ref[...]
    h = jnp.maximum(
        jnp.dot(x, Wg1_ref[...], preferred_element_type=jnp.float32)
        + bg1_ref[...],
        0.0,
    )
    logits = (
        jnp.dot(h, Wg2_ref[...], preferred_element_type=jnp.float32)
        + bg2_ref[...]
    )
    m = jnp.max(logits, axis=1, keepdims=True)
    ex = jnp.exp(logits - m)
    probs = ex / jnp.sum(ex, axis=1, keepdims=True)
    probs_ref[...] = probs

    ii = jax.lax.broadcasted_iota(jnp.int32, (B, E), 1)
    m1 = jnp.max(probs, axis=1, keepdims=True)
    i1 = jnp.min(jnp.where(probs == m1, ii, E), axis=1, keepdims=True)
    masked = jnp.where(ii == i1, -1.0, probs)
    m2 = jnp.max(masked, axis=1, keepdims=True)
    i2 = jnp.min(jnp.where(masked == m2, ii, E), axis=1, keepdims=True)
    w_ref[...] = jnp.where((ii == i1) | (ii == i2), probs, 0.0)


def _ffn_body(x_ref, We1_ref, be1_ref, We2_ref, be2_ref, w_ref, acc_ref):
    e = pl.program_id(0)
    x = x_ref[...]
    h = jnp.maximum(
        jnp.dot(x, We1_ref[0], preferred_element_type=jnp.float32)
        + be1_ref[0],
        0.0,
    )
    y = (
        jnp.dot(h, We2_ref[0], preferred_element_type=jnp.float32)
        + be2_ref[0]
    )
    ii = jax.lax.broadcasted_iota(jnp.int32, (B, E), 1)
    col = jnp.sum(
        jnp.where(ii == e, w_ref[...], 0.0), axis=1, keepdims=True
    )
    contrib = col * y

    @pl.when(e == 0)
    def _():
        acc_ref[...] = contrib

    @pl.when(e != 0)
    def _():
        acc_ref[...] += contrib


def _head_body(c_ref, Wc1_ref, bc1_ref, Wc2_ref, bc2_ref, out_ref):
    h = jnp.maximum(
        jnp.dot(c_ref[...], Wc1_ref[...], preferred_element_type=jnp.float32)
        + bc1_ref[...],
        0.0,
    )
    out_ref[...] = (
        jnp.dot(h, Wc2_ref[...], preferred_element_type=jnp.float32)
        + bc2_ref[...]
    )


def kernel(x, Wg1, bg1, Wg2, bg2, We1, be1, We2, be2, Wc1, bc1, Wc2, bc2):
    probs, w = pl.pallas_call(
        _gating_body,
        out_shape=(
            jax.ShapeDtypeStruct((B, E), jnp.float32),
            jax.ShapeDtypeStruct((B, E), jnp.float32),
        ),
    )(x, Wg1, bg1.reshape(1, GH), Wg2, bg2.reshape(1, E))

    combined = pl.pallas_call(
        _ffn_body,
        grid=(E,),
        in_specs=[
            pl.BlockSpec((B, D), lambda e: (0, 0)),
            pl.BlockSpec((1, D, DFF), lambda e: (e, 0, 0)),
            pl.BlockSpec((1, 1, DFF), lambda e: (e, 0, 0)),
            pl.BlockSpec((1, DFF, D), lambda e: (e, 0, 0)),
            pl.BlockSpec((1, 1, D), lambda e: (e, 0, 0)),
            pl.BlockSpec((B, E), lambda e: (0, 0)),
        ],
        out_specs=pl.BlockSpec((B, D), lambda e: (0, 0)),
        out_shape=jax.ShapeDtypeStruct((B, D), jnp.float32),
    )(x, We1, be1.reshape(E, 1, DFF), We2, be2.reshape(E, 1, D), w)

    logits = pl.pallas_call(
        _head_body,
        out_shape=jax.ShapeDtypeStruct((B, OUT), jnp.float32),
    )(combined, Wc1, bc1.reshape(1, CH), Wc2, bc2.reshape(1, OUT))

    return (logits, probs)
